# Initial kernel scaffold; baseline (speedup 1.0000x reference)
import jax
import jax.numpy as jnp
from jax.experimental import pallas as pl


def kernel(x, emb, W, b):
    raise NotImplementedError("write your pallas kernel here")


# speedup vs baseline: 4.9909x; 4.9909x over previous
import functools

import jax
import jax.numpy as jnp
from jax import lax
from jax.experimental import pallas as pl
from jax.experimental.pallas import tpu as pltpu
from jax.experimental.pallas import tpu_sc as plsc

VOCAB = 1000
DIM = 64
SENT = 50
B = 4096
OUT = SENT + 1

try:
    _info = plsc.get_sparse_core_info()
    _NC, _NS, _L = _info.num_cores, _info.num_subcores, _info.num_lanes
except Exception:
    _NC, _NS, _L = 2, 16, 16

NW = _NC * _NS
BPW = B // NW
NGROUPS = BPW // _L
NCB = DIM // _L

_mesh = plsc.VectorSubcoreMesh(
    core_axis_name="c", subcore_axis_name="s",
    num_cores=_NC, num_subcores=_NS,
)


@functools.partial(
    pl.kernel,
    out_type=jax.ShapeDtypeStruct((B, DIM), jnp.float32),
    mesh=_mesh,
    scratch_types=[
        pltpu.VMEM((VOCAB * DIM,), jnp.float32),
        pltpu.VMEM((SENT * BPW,), jnp.int32),
        pltpu.VMEM((BPW, DIM), jnp.float32),
        pltpu.SemaphoreType.DMA,
    ],
    compiler_params=pltpu.CompilerParams(needs_layout_passes=False),
)
def _sc_pool(emb_hbm, xw_hbm, out_hbm, table_v, idx_v, pool_v, sem):
    w = lax.axis_index("s") * _NC + lax.axis_index("c")
    base = w * BPW
    table_cp = pltpu.async_copy(emb_hbm, table_v, sem)
    pltpu.sync_copy(xw_hbm.at[pl.ds(w * SENT * BPW, SENT * BPW)], idx_v)
    table_cp.wait()

    lane = lax.iota(jnp.int32, _L)
    for g in range(NGROUPS):
        b_lane = lane + (g * _L)
        for cb in range(NCB):
            col0 = cb * _L

            def body(t, accs, g=g, col0=col0):
                rows = idx_v[pl.ds(t * BPW + g * _L, _L)]
                fbase = rows * DIM + col0
                return tuple(
                    accs[c] + plsc.load_gather(table_v, [fbase + c])
                    for c in range(_L)
                )

            accs = lax.fori_loop(
                0, SENT, body,
                tuple(jnp.zeros((_L,), jnp.float32) for _ in range(_L)),
            )
            for c in range(_L):
                plsc.store_scatter(
                    pool_v,
                    [b_lane, jnp.full((_L,), col0 + c, jnp.int32)],
                    accs[c],
                )
    pltpu.sync_copy(pool_v, out_hbm.at[pl.ds(base, BPW)])


def _head_body(p_ref, wt_ref, b_ref, o_ref):
    logits = jnp.dot(p_ref[...], wt_ref[...],
                     preferred_element_type=jnp.float32)
    logits = logits + b_ref[...]
    m = jnp.max(logits, axis=-1, keepdims=True)
    e = jnp.exp(logits - m)
    o_ref[...] = e * (1.0 / jnp.sum(e, axis=-1, keepdims=True))


_HEAD_BLOCK = 512
_head = pl.pallas_call(
    _head_body,
    grid=(B // _HEAD_BLOCK,),
    in_specs=[
        pl.BlockSpec((_HEAD_BLOCK, DIM), lambda i: (i, 0)),
        pl.BlockSpec((DIM, OUT), lambda i: (0, 0)),
        pl.BlockSpec((1, OUT), lambda i: (0, 0)),
    ],
    out_specs=pl.BlockSpec((_HEAD_BLOCK, OUT), lambda i: (i, 0)),
    out_shape=jax.ShapeDtypeStruct((B, OUT), jnp.float32),
)


def kernel(x, emb, W, b):
    xw = (x.astype(jnp.int32).T
          .reshape(SENT, NW, BPW).transpose(1, 0, 2).reshape(NW * SENT * BPW))
    pooled_sum = _sc_pool(emb.reshape(-1), xw)
    wt = W.T.astype(jnp.float32) * (1.0 / SENT)
    return _head(pooled_sum, wt, b.reshape(1, OUT))


# speedup vs baseline: 13.8142x; 2.7679x over previous
import functools

import jax
import jax.numpy as jnp
from jax import lax
from jax.experimental import pallas as pl
from jax.experimental.pallas import tpu as pltpu
from jax.experimental.pallas import tpu_sc as plsc

VOCAB = 1000
DIM = 64
SENT = 50
B = 4096
OUT = SENT + 1
TPAD = 64

try:
    _info = plsc.get_sparse_core_info()
    _NC, _NS, _L = _info.num_cores, _info.num_subcores, _info.num_lanes
except Exception:
    _NC, _NS, _L = 2, 16, 16

NW = _NC * _NS
BPW = B // NW
NTG = TPAD // _L
NCB = DIM // _L

_mesh = plsc.VectorSubcoreMesh(
    core_axis_name="c", subcore_axis_name="s",
    num_cores=_NC, num_subcores=_NS,
)

@functools.partial(
    pl.kernel,
    out_type=jax.ShapeDtypeStruct((B * DIM,), jnp.float32),
    mesh=_mesh,
    scratch_types=[
        pltpu.VMEM((VOCAB * DIM,), jnp.float32),
        pltpu.VMEM((BPW * TPAD,), jnp.int32),
        pltpu.VMEM((BPW * DIM,), jnp.float32),
        pltpu.SemaphoreType.DMA,
    ],
    compiler_params=pltpu.CompilerParams(needs_layout_passes=False),
)
def _sc_pool(emb_hbm, xp_hbm, out_hbm, table_v, idx_v, pool_v, sem):
    w = lax.axis_index("s") * _NC + lax.axis_index("c")
    table_cp = pltpu.async_copy(emb_hbm, table_v, sem)
    pltpu.sync_copy(xp_hbm.at[pl.ds(w * (BPW * TPAD), BPW * TPAD)], idx_v)
    table_cp.wait()

    def body(b, carry):
        bt = b * TPAD
        accs = [jnp.zeros((_L,), jnp.float32) for _ in range(NCB)]
        for tg in range(NTG):
            toks = idx_v[pl.ds(bt + tg * _L, _L)]
            for j in range(_L):
                base = toks[j] * DIM
                for k in range(NCB):
                    accs[k] = accs[k] + table_v[pl.ds(base + k * _L, _L)]
        for k in range(NCB):
            pool_v[pl.ds(b * DIM + k * _L, _L)] = accs[k]
        return carry

    lax.fori_loop(0, BPW, body, jnp.int32(0))
    pltpu.sync_copy(pool_v, out_hbm.at[pl.ds(w * (BPW * DIM), BPW * DIM)])


def _head_body(p_ref, wt_ref, b_ref, o_ref):
    logits = jnp.dot(p_ref[...], wt_ref[...],
                     preferred_element_type=jnp.float32)
    logits = logits + b_ref[...]
    m = jnp.max(logits, axis=-1, keepdims=True)
    e = jnp.exp(logits - m)
    o_ref[...] = e * (1.0 / jnp.sum(e, axis=-1, keepdims=True))


_HEAD_BLOCK = 512
_head = pl.pallas_call(
    _head_body,
    grid=(B // _HEAD_BLOCK,),
    in_specs=[
        pl.BlockSpec((_HEAD_BLOCK, DIM), lambda i: (i, 0)),
        pl.BlockSpec((DIM, OUT), lambda i: (0, 0)),
        pl.BlockSpec((1, OUT), lambda i: (0, 0)),
    ],
    out_specs=pl.BlockSpec((_HEAD_BLOCK, OUT), lambda i: (i, 0)),
    out_shape=jax.ShapeDtypeStruct((B, OUT), jnp.float32),
)


def kernel(x, emb, W, b):
    xp = jnp.pad(x.astype(jnp.int32), ((0, 0), (0, TPAD - SENT))).reshape(-1)
    pooled_sum = _sc_pool(emb.reshape(-1), xp)
    pooled_sum = pooled_sum.reshape(B, DIM)
    wt = W.T.astype(jnp.float32) * (1.0 / SENT)
    return _head(pooled_sum, wt, b.reshape(1, OUT))
